# Initial kernel scaffold; baseline (speedup 1.0000x reference)
import jax
import jax.numpy as jnp
from jax.experimental import pallas as pl


def kernel(feats, offsets, ln_g, ln_b, W1, b1, W2, b2):
    raise NotImplementedError("write your pallas kernel here")


# speedup vs baseline: 13.5380x; 13.5380x over previous
import jax
import jax.numpy as jnp
from jax import lax
from jax.experimental import pallas as pl
from jax.experimental.pallas import tpu as pltpu


def _pick_block(n):
    for t in (2048, 1024, 512, 256, 128, 64, 32, 16, 8):
        if n % t == 0:
            return t
    return n


def kernel(feats, offsets, ln_g, ln_b, W1, b1, W2, b2):
    N, D = feats.shape
    B = offsets.shape[0] - 1
    H = W1.shape[1]
    T = _pick_block(N)
    K = N // T

    starts = offsets[:-1].reshape(B, 1).astype(jnp.int32)
    ends = offsets[1:].reshape(B, 1).astype(jnp.int32)
    Wg = ln_g[:, None] * W1
    csum = jnp.sum(Wg, axis=0).reshape(1, H)
    c1 = (ln_b @ W1 + b1).reshape(1, H)
    W2_r = W2.reshape(1, H)
    b2_r = b2.reshape(1, 1)

    def body(x_ref, st_ref, en_ref, Wg_ref, cs_ref, c1_ref, W2_ref,
             b2_ref, out_ref, m_ref, s_ref, acc_ref):
        i = pl.program_id(0)

        @pl.when(i == 0)
        def _init():
            m_ref[...] = jnp.full_like(m_ref, -jnp.inf)
            s_ref[...] = jnp.zeros_like(s_ref)
            acc_ref[...] = jnp.zeros_like(acc_ref)

        x = x_ref[...]
        mu = jnp.mean(x, axis=1, keepdims=True)
        ms = jnp.mean(x * x, axis=1, keepdims=True)
        var = jnp.maximum(ms - mu * mu, 0.0)
        r = lax.rsqrt(var + 1e-5)
        xw = jnp.dot(x, Wg_ref[...],
                     preferred_element_type=jnp.float32)
        h = r * xw - (r * mu) * cs_ref[...] + c1_ref[...]
        h = 0.5 * h * (1.0 + lax.erf(h * 0.7071067811865476))
        w_row = lax.dot_general(W2_ref[...], h, (((1,), (1,)), ((), ())),
                                preferred_element_type=jnp.float32)
        w_row = w_row + b2_ref[...]

        gidx = i * T + lax.broadcasted_iota(jnp.int32, (B, T), 1)
        mask = (gidx >= st_ref[...]) & (gidx < en_ref[...])

        wneg = jnp.where(mask, w_row, -jnp.inf)
        bmax = jnp.max(wneg, axis=1, keepdims=True)
        m_old = m_ref[...]
        m_new = jnp.maximum(m_old, bmax)
        m_safe = jnp.where(m_new > -jnp.inf, m_new, 0.0)
        e = jnp.where(mask, jnp.exp(w_row - m_safe), 0.0)
        scale = jnp.where(m_old > -jnp.inf, jnp.exp(m_old - m_new), 0.0)

        s_ref[...] = s_ref[...] * scale + jnp.sum(e, axis=1, keepdims=True)
        acc_ref[...] = acc_ref[...] * scale + jnp.dot(
            e, x, preferred_element_type=jnp.float32)
        m_ref[...] = m_new

        @pl.when(i == pl.num_programs(0) - 1)
        def _fin():
            s = s_ref[...]
            out_ref[...] = acc_ref[...] / jnp.where(s > 0, s, 1.0)

    out = pl.pallas_call(
        body,
        grid=(K,),
        in_specs=[
            pl.BlockSpec((T, D), lambda i: (i, 0)),
            pl.BlockSpec((B, 1), lambda i: (0, 0)),
            pl.BlockSpec((B, 1), lambda i: (0, 0)),
            pl.BlockSpec((D, H), lambda i: (0, 0)),
            pl.BlockSpec((1, H), lambda i: (0, 0)),
            pl.BlockSpec((1, H), lambda i: (0, 0)),
            pl.BlockSpec((1, H), lambda i: (0, 0)),
            pl.BlockSpec((1, 1), lambda i: (0, 0)),
        ],
        out_specs=pl.BlockSpec((B, D), lambda i: (0, 0)),
        out_shape=jax.ShapeDtypeStruct((B, D), jnp.float32),
        scratch_shapes=[
            pltpu.VMEM((B, 1), jnp.float32),
            pltpu.VMEM((B, 1), jnp.float32),
            pltpu.VMEM((B, D), jnp.float32),
        ],
    )(feats, starts, ends, Wg, csum, c1, W2_r, b2_r)
    return out


# speedup vs baseline: 15.1701x; 1.1206x over previous
import jax
import jax.numpy as jnp
from jax import lax
from jax.experimental import pallas as pl
from jax.experimental.pallas import tpu as pltpu


def _pick_block(n):
    for t in (4096, 2048, 1024, 512, 256, 128, 64, 32, 16, 8):
        if n % t == 0:
            return t
    return n


def kernel(feats, offsets, ln_g, ln_b, W1, b1, W2, b2):
    N, D = feats.shape
    B = offsets.shape[0] - 1
    H = W1.shape[1]
    T = _pick_block(N)
    K = N // T

    starts = offsets[:-1].reshape(B, 1).astype(jnp.int32)
    ends = offsets[1:].reshape(B, 1).astype(jnp.int32)
    Wg = ln_g[:, None] * W1
    csum = jnp.sum(Wg, axis=0).reshape(1, H)
    c1 = (ln_b @ W1 + b1).reshape(1, H)
    W2_r = W2.reshape(1, H)
    b2_r = b2.reshape(1, 1)

    def body(x_ref, st_ref, en_ref, Wg_ref, cs_ref, c1_ref, W2_ref,
             b2_ref, out_ref, m_ref, s_ref, acc_ref):
        i = pl.program_id(0)

        @pl.when(i == 0)
        def _init():
            m_ref[...] = jnp.full_like(m_ref, -jnp.inf)
            s_ref[...] = jnp.zeros_like(s_ref)
            acc_ref[...] = jnp.zeros_like(acc_ref)

        x = x_ref[...]
        mu = jnp.mean(x, axis=1, keepdims=True)
        ms = jnp.mean(x * x, axis=1, keepdims=True)
        var = jnp.maximum(ms - mu * mu, 0.0)
        r = lax.rsqrt(var + 1e-5)
        xw = jnp.dot(x, Wg_ref[...],
                     preferred_element_type=jnp.float32)
        h = r * xw - (r * mu) * cs_ref[...] + c1_ref[...]
        h = 0.5 * h * (1.0 + lax.erf(h * 0.7071067811865476))
        w_row = lax.dot_general(W2_ref[...], h, (((1,), (1,)), ((), ())),
                                preferred_element_type=jnp.float32)
        w_row = w_row + b2_ref[...]

        gidx = i * T + lax.broadcasted_iota(jnp.int32, (B, T), 1)
        mask = (gidx >= st_ref[...]) & (gidx < en_ref[...])

        wneg = jnp.where(mask, w_row, -jnp.inf)
        bmax = jnp.max(wneg, axis=1, keepdims=True)
        m_old = m_ref[...]
        m_new = jnp.maximum(m_old, bmax)
        m_safe = jnp.where(m_new > -jnp.inf, m_new, 0.0)
        e = jnp.where(mask, jnp.exp(w_row - m_safe), 0.0)
        scale = jnp.where(m_old > -jnp.inf, jnp.exp(m_old - m_new), 0.0)

        s_ref[...] = s_ref[...] * scale + jnp.sum(e, axis=1, keepdims=True)
        acc_ref[...] = acc_ref[...] * scale + jnp.dot(
            e, x, preferred_element_type=jnp.float32)
        m_ref[...] = m_new

        @pl.when(i == pl.num_programs(0) - 1)
        def _fin():
            s = s_ref[...]
            out_ref[...] = acc_ref[...] / jnp.where(s > 0, s, 1.0)

    out = pl.pallas_call(
        body,
        grid=(K,),
        in_specs=[
            pl.BlockSpec((T, D), lambda i: (i, 0)),
            pl.BlockSpec((B, 1), lambda i: (0, 0)),
            pl.BlockSpec((B, 1), lambda i: (0, 0)),
            pl.BlockSpec((D, H), lambda i: (0, 0)),
            pl.BlockSpec((1, H), lambda i: (0, 0)),
            pl.BlockSpec((1, H), lambda i: (0, 0)),
            pl.BlockSpec((1, H), lambda i: (0, 0)),
            pl.BlockSpec((1, 1), lambda i: (0, 0)),
        ],
        out_specs=pl.BlockSpec((B, D), lambda i: (0, 0)),
        out_shape=jax.ShapeDtypeStruct((B, D), jnp.float32),
        scratch_shapes=[
            pltpu.VMEM((B, 1), jnp.float32),
            pltpu.VMEM((B, 1), jnp.float32),
            pltpu.VMEM((B, D), jnp.float32),
        ],
    )(feats, starts, ends, Wg, csum, c1, W2_r, b2_r)
    return out


# speedup vs baseline: 15.5128x; 1.0226x over previous
import jax
import jax.numpy as jnp
from jax import lax
from jax.experimental import pallas as pl
from jax.experimental.pallas import tpu as pltpu


def _pick_block(n):
    for t in (8192, 4096, 2048, 1024, 512, 256, 128, 64, 32, 16, 8):
        if n % t == 0:
            return t
    return n


def kernel(feats, offsets, ln_g, ln_b, W1, b1, W2, b2):
    N, D = feats.shape
    B = offsets.shape[0] - 1
    H = W1.shape[1]
    T = _pick_block(N)
    K = N // T

    starts = offsets[:-1].reshape(B, 1).astype(jnp.int32)
    ends = offsets[1:].reshape(B, 1).astype(jnp.int32)
    Wg = ln_g[:, None] * W1
    csum = jnp.sum(Wg, axis=0).reshape(1, H)
    c1 = (ln_b @ W1 + b1).reshape(1, H)
    W2_r = W2.reshape(1, H)
    b2_r = b2.reshape(1, 1)

    def body(x_ref, st_ref, en_ref, Wg_ref, cs_ref, c1_ref, W2_ref,
             b2_ref, out_ref, m_ref, s_ref, acc_ref):
        i = pl.program_id(0)

        @pl.when(i == 0)
        def _init():
            m_ref[...] = jnp.full_like(m_ref, -jnp.inf)
            s_ref[...] = jnp.zeros_like(s_ref)
            acc_ref[...] = jnp.zeros_like(acc_ref)

        x = x_ref[...]
        mu = jnp.mean(x, axis=1, keepdims=True)
        ms = jnp.mean(x * x, axis=1, keepdims=True)
        var = jnp.maximum(ms - mu * mu, 0.0)
        r = lax.rsqrt(var + 1e-5)
        xw = jnp.dot(x, Wg_ref[...],
                     preferred_element_type=jnp.float32)
        h = r * xw - (r * mu) * cs_ref[...] + c1_ref[...]
        h = 0.5 * h * (1.0 + lax.erf(h * 0.7071067811865476))
        w_row = lax.dot_general(W2_ref[...], h, (((1,), (1,)), ((), ())),
                                preferred_element_type=jnp.float32)
        w_row = w_row + b2_ref[...]

        gidx = i * T + lax.broadcasted_iota(jnp.int32, (B, T), 1)
        mask = (gidx >= st_ref[...]) & (gidx < en_ref[...])

        wneg = jnp.where(mask, w_row, -jnp.inf)
        bmax = jnp.max(wneg, axis=1, keepdims=True)
        m_old = m_ref[...]
        m_new = jnp.maximum(m_old, bmax)
        m_safe = jnp.where(m_new > -jnp.inf, m_new, 0.0)
        e = jnp.exp(wneg - m_safe)
        scale = jnp.where(m_old > -jnp.inf, jnp.exp(m_old - m_new), 0.0)

        s_ref[...] = s_ref[...] * scale + jnp.sum(e, axis=1, keepdims=True)
        acc_ref[...] = acc_ref[...] * scale + jnp.dot(
            e, x, preferred_element_type=jnp.float32)
        m_ref[...] = m_new

        @pl.when(i == pl.num_programs(0) - 1)
        def _fin():
            s = s_ref[...]
            out_ref[...] = acc_ref[...] / jnp.where(s > 0, s, 1.0)

    out = pl.pallas_call(
        body,
        grid=(K,),
        in_specs=[
            pl.BlockSpec((T, D), lambda i: (i, 0)),
            pl.BlockSpec((B, 1), lambda i: (0, 0)),
            pl.BlockSpec((B, 1), lambda i: (0, 0)),
            pl.BlockSpec((D, H), lambda i: (0, 0)),
            pl.BlockSpec((1, H), lambda i: (0, 0)),
            pl.BlockSpec((1, H), lambda i: (0, 0)),
            pl.BlockSpec((1, H), lambda i: (0, 0)),
            pl.BlockSpec((1, 1), lambda i: (0, 0)),
        ],
        out_specs=pl.BlockSpec((B, D), lambda i: (0, 0)),
        out_shape=jax.ShapeDtypeStruct((B, D), jnp.float32),
        scratch_shapes=[
            pltpu.VMEM((B, 1), jnp.float32),
            pltpu.VMEM((B, 1), jnp.float32),
            pltpu.VMEM((B, D), jnp.float32),
        ],
    )(feats, starts, ends, Wg, csum, c1, W2_r, b2_r)
    return out
